# baseline (speedup 1.0000x reference)
import functools

import jax
import jax.numpy as jnp
from jax import lax
from jax.experimental import pallas as pl
from jax.experimental.pallas import tpu as pltpu
from jax.experimental.pallas import tpu_sc as plsc

EPS = 1e-12


def _ln_table_kernel(t_ref, g_ref, b_ref, o_ref):
    t = t_ref[...]
    m = jnp.mean(t, axis=1, keepdims=True)
    c = t - m
    v = jnp.mean(c * c, axis=1, keepdims=True)
    o_ref[...] = c * lax.rsqrt(v + EPS) * g_ref[...] + b_ref[...]


def _normalize_table(table, gamma, beta):
    V, D = table.shape
    return pl.pallas_call(
        _ln_table_kernel,
        out_shape=jax.ShapeDtypeStruct((V, D), jnp.float32),
    )(table, gamma.reshape(1, D), beta.reshape(1, D))


@functools.lru_cache(maxsize=None)
def _make_gather(V, D, N):
    info = plsc.get_sparse_core_info()
    NC, NS, L = info.num_cores, info.num_subcores, info.num_lanes
    NW = NC * NS
    H = D // 2
    CHUNK = 128
    NSLAB = NW // 2
    assert N % (NSLAB * CHUNK) == 0
    n_chunks = N // (NSLAB * CHUNK)
    NBUF = 2
    GROUPS = CHUNK // L

    mesh = plsc.VectorSubcoreMesh(core_axis_name="c", subcore_axis_name="s")

    @functools.partial(
        pl.kernel,
        mesh=mesh,
        out_type=jax.ShapeDtypeStruct((N, 2, H), jnp.float32),
        scratch_types=[
            pltpu.VMEM((V * H,), jnp.float32),
            pltpu.VMEM((NBUF, CHUNK), jnp.int32),
            pltpu.VMEM((NBUF, CHUNK, 1, H), jnp.float32),
            pltpu.SemaphoreType.DMA,
            pltpu.SemaphoreType.DMA,
        ],
        compiler_params=pltpu.CompilerParams(needs_layout_passes=False),
    )
    def gather(idx_hbm, tlo_hbm, thi_hbm, out_hbm, tab_v, idx_v, rows_v, isem, ssem):
        wid = lax.axis_index("s") * NC + lax.axis_index("c")
        h = lax.rem(wid, 2)
        slab = wid // 2
        base = slab * (n_chunks * CHUNK)

        @pl.when(h == 0)
        def _():
            pltpu.sync_copy(tlo_hbm, tab_v)

        @pl.when(h == 1)
        def _():
            pltpu.sync_copy(thi_hbm, tab_v)

        def fire_idx(j, buf):
            return pltpu.async_copy(
                idx_hbm.at[pl.ds(base + j * CHUNK, CHUNK)], idx_v.at[buf], isem
            )

        def wait_idx(buf):
            pltpu.make_async_copy(
                idx_hbm.at[pl.ds(base, CHUNK)], idx_v.at[buf], isem
            ).wait()

        def drain_out():
            pltpu.make_async_copy(
                rows_v.at[0],
                out_hbm.at[pl.ds(base, CHUNK), pl.ds(0, 1)],
                ssem,
            ).wait()

        lanes = lax.iota(jnp.int32, L)
        zero = jnp.zeros((L,), jnp.int32)

        def assemble(j, buf):
            obuf = rows_v.at[buf]
            for g in range(GROUPS):
                idH = idx_v[buf, pl.ds(g * L, L)] * H
                tokv = lanes + (g * L)
                for c in range(H):
                    col = plsc.load_gather(tab_v, [idH + c])
                    plsc.store_scatter(
                        obuf, [tokv, zero, jnp.full((L,), c, jnp.int32)], col
                    )

        for j0 in range(NBUF):
            fire_idx(j0, j0)

        def body(j, _):
            buf = lax.rem(j, NBUF)
            wait_idx(buf)

            @pl.when(j >= NBUF)
            def _():
                drain_out()

            assemble(j, buf)
            pltpu.async_copy(
                rows_v.at[buf],
                out_hbm.at[pl.ds(base + j * CHUNK, CHUNK), pl.ds(h, 1)],
                ssem,
            )

            @pl.when(j + NBUF < n_chunks)
            def _():
                fire_idx(j + NBUF, buf)

            return 0

        lax.fori_loop(0, n_chunks, body, 0)
        for _ in range(NBUF):
            drain_out()

    return gather


def kernel(input_ids, table, gamma, beta):
    B, L = input_ids.shape
    V, D = table.shape
    N = B * L
    normed = _normalize_table(table, gamma, beta)
    tlo = normed[:, : D // 2].reshape(-1)
    thi = normed[:, D // 2 :].reshape(-1)
    idx = input_ids.reshape(N).astype(jnp.int32)
    out = _make_gather(V, D, N)(idx, tlo, thi)
    return out.reshape(B, L, D)


# speedup vs baseline: 1.7425x; 1.7425x over previous
import functools

import jax
import jax.numpy as jnp
from jax import lax
from jax.experimental import pallas as pl
from jax.experimental.pallas import tpu as pltpu
from jax.experimental.pallas import tpu_sc as plsc

EPS = 1e-12


def _ln_table_kernel(t_ref, g_ref, b_ref, o_ref):
    t = t_ref[...]
    m = jnp.mean(t, axis=1, keepdims=True)
    c = t - m
    v = jnp.mean(c * c, axis=1, keepdims=True)
    o_ref[...] = c * lax.rsqrt(v + EPS) * g_ref[...] + b_ref[...]


def _normalize_table(table, gamma, beta):
    V, D = table.shape
    return pl.pallas_call(
        _ln_table_kernel,
        out_shape=jax.ShapeDtypeStruct((V, D), jnp.float32),
    )(table, gamma.reshape(1, D), beta.reshape(1, D))


@functools.lru_cache(maxsize=None)
def _make_gather(V, D, N):
    info = plsc.get_sparse_core_info()
    NC, NS, L = info.num_cores, info.num_subcores, info.num_lanes
    NW = NC * NS
    H = D // 2
    CHUNK = 128
    NSLAB = NW // 2
    assert N % (NSLAB * CHUNK) == 0
    n_chunks = N // (NSLAB * CHUNK)
    NBUF = 2
    GROUPS = CHUNK // L

    mesh = plsc.VectorSubcoreMesh(core_axis_name="c", subcore_axis_name="s")

    @functools.partial(
        pl.kernel,
        mesh=mesh,
        out_type=jax.ShapeDtypeStruct((N, 2, H), jnp.float32),
        scratch_types=[
            pltpu.VMEM((V * H,), jnp.float32),
            pltpu.VMEM((NBUF, CHUNK), jnp.int32),
            pltpu.VMEM((NBUF, CHUNK, 1, H), jnp.float32),
            pltpu.SemaphoreType.DMA,
            pltpu.SemaphoreType.DMA,
        ],
        compiler_params=pltpu.CompilerParams(needs_layout_passes=False),
    )
    def gather(idx_hbm, tlo_hbm, thi_hbm, out_hbm, tab_v, idx_v, rows_v, isem, ssem):
        wid = lax.axis_index("s") * NC + lax.axis_index("c")
        h = lax.rem(wid, 2)
        slab = wid // 2
        base = slab * (n_chunks * CHUNK)

        @pl.when(h == 0)
        def _():
            pltpu.sync_copy(tlo_hbm, tab_v)

        @pl.when(h == 1)
        def _():
            pltpu.sync_copy(thi_hbm, tab_v)

        def fire_idx(j, buf):
            return pltpu.async_copy(
                idx_hbm.at[pl.ds(base + j * CHUNK, CHUNK)], idx_v.at[buf], isem
            )

        def wait_idx(buf):
            pltpu.make_async_copy(
                idx_hbm.at[pl.ds(base, CHUNK)], idx_v.at[buf], isem
            ).wait()

        def drain_out():
            pltpu.make_async_copy(
                rows_v.at[0],
                out_hbm.at[pl.ds(base, CHUNK), pl.ds(0, 1)],
                ssem,
            ).wait()

        lanes = lax.iota(jnp.int32, L)

        def assemble(j, buf):
            obuf = rows_v.at[buf]
            ibuf = idx_v.at[buf]
            for t in range(CHUNK):
                a0 = plsc.load_gather(ibuf, [jnp.full((L,), t, jnp.int32)])
                a0 = a0 * H + lanes
                for k in range(H // L):
                    row = plsc.load_gather(tab_v, [a0 + (k * L)])
                    obuf[t, 0, pl.ds(k * L, L)] = row

        for j0 in range(NBUF):
            fire_idx(j0, j0)

        def body(j, _):
            buf = lax.rem(j, NBUF)
            wait_idx(buf)

            @pl.when(j >= NBUF)
            def _():
                drain_out()

            assemble(j, buf)
            pltpu.async_copy(
                rows_v.at[buf],
                out_hbm.at[pl.ds(base + j * CHUNK, CHUNK), pl.ds(h, 1)],
                ssem,
            )

            @pl.when(j + NBUF < n_chunks)
            def _():
                fire_idx(j + NBUF, buf)

            return 0

        lax.fori_loop(0, n_chunks, body, 0)
        for _ in range(NBUF):
            drain_out()

    return gather


def kernel(input_ids, table, gamma, beta):
    B, L = input_ids.shape
    V, D = table.shape
    N = B * L
    normed = _normalize_table(table, gamma, beta)
    tlo = normed[:, : D // 2].reshape(-1)
    thi = normed[:, D // 2 :].reshape(-1)
    idx = input_ids.reshape(N).astype(jnp.int32)
    out = _make_gather(V, D, N)(idx, tlo, thi)
    return out.reshape(B, L, D)


# speedup vs baseline: 13.2257x; 7.5903x over previous
import functools

import jax
import jax.numpy as jnp
from jax import lax
from jax.experimental import pallas as pl
from jax.experimental.pallas import tpu as pltpu
from jax.experimental.pallas import tpu_sc as plsc

EPS = 1e-12


def _ln_table_kernel(t_ref, g_ref, b_ref, o_ref):
    t = t_ref[...]
    m = jnp.mean(t, axis=1, keepdims=True)
    c = t - m
    v = jnp.mean(c * c, axis=1, keepdims=True)
    o_ref[...] = c * lax.rsqrt(v + EPS) * g_ref[...] + b_ref[...]


def _normalize_table(table, gamma, beta):
    V, D = table.shape
    return pl.pallas_call(
        _ln_table_kernel,
        out_shape=jax.ShapeDtypeStruct((V, D), jnp.float32),
    )(table, gamma.reshape(1, D), beta.reshape(1, D))


@functools.lru_cache(maxsize=None)
def _make_gather(V, D, N):
    info = plsc.get_sparse_core_info()
    NC, NS = info.num_cores, info.num_subcores
    NW = NC * NS
    CHUNK = 128
    assert N % (NW * CHUNK) == 0
    n_chunks = N // (NW * CHUNK)
    NBUF = 6
    GA = NBUF - 1

    mesh = plsc.VectorSubcoreMesh(core_axis_name="c", subcore_axis_name="s")

    @functools.partial(
        pl.kernel,
        mesh=mesh,
        out_type=jax.ShapeDtypeStruct((N, D), jnp.float32),
        scratch_types=[
            pltpu.VMEM((n_chunks, CHUNK), jnp.int32),
            pltpu.VMEM((NBUF, CHUNK, D), jnp.float32),
            pltpu.SemaphoreType.DMA,
            pltpu.SemaphoreType.DMA,
        ],
    )
    def gather(idx_hbm, tab_hbm, out_hbm, idx_v, rows_v, gsem, ssem):
        wid = lax.axis_index("s") * NC + lax.axis_index("c")
        base = wid * (n_chunks * CHUNK)
        pltpu.sync_copy(idx_hbm.at[wid], idx_v)

        def fire(j, buf):
            return pltpu.async_copy(tab_hbm.at[idx_v.at[j]], rows_v.at[buf], gsem)

        def drain_one_scatter():
            pltpu.make_async_copy(
                rows_v.at[0],
                out_hbm.at[pl.ds(base, CHUNK)],
                ssem,
            ).wait()

        for j0 in range(GA):
            fire(j0, j0)

        def body(j, _):
            buf = lax.rem(j, NBUF)

            @pl.when(j + GA < n_chunks)
            def _():
                @pl.when(j >= 1)
                def _():
                    drain_one_scatter()

                fire(j + GA, lax.rem(j + GA, NBUF))

            pltpu.make_async_copy(
                tab_hbm.at[idx_v.at[j]], rows_v.at[buf], gsem
            ).wait()
            pltpu.async_copy(
                rows_v.at[buf],
                out_hbm.at[pl.ds(base + j * CHUNK, CHUNK)],
                ssem,
            )
            return 0

        lax.fori_loop(0, n_chunks, body, 0)
        for _ in range(NBUF):
            drain_one_scatter()

    return gather


def kernel(input_ids, table, gamma, beta):
    B, L = input_ids.shape
    V, D = table.shape
    N = B * L
    normed = _normalize_table(table, gamma, beta)
    info = plsc.get_sparse_core_info()
    NW = info.num_cores * info.num_subcores
    idx = input_ids.reshape(NW, N // (NW * 128), 128).astype(jnp.int32)
    out = _make_gather(V, D, N)(idx, normed)
    return out.reshape(B, L, D)
